# Initial kernel scaffold; baseline (speedup 1.0000x reference)
import jax
import jax.numpy as jnp
from jax.experimental import pallas as pl


def kernel(x, pos, edge_index, params):
    raise NotImplementedError("write your pallas kernel here")


# speedup vs baseline: 1.2344x; 1.2344x over previous
import functools
import jax
import jax.numpy as jnp
from jax.experimental import pallas as pl
from jax.experimental.pallas import tpu as pltpu

N = 10000
E = 320000
C = 128
SP = 8
CA = C // SP
EPS = 1e-5


def _pre_body(x_ref, win_ref, g1_ref, b1_ref, wsrc_ref, bsrc_ref,
              wdst_ref, bdst_ref, wlin_ref, blin_ref,
              asrc_ref, adst_ref, xl_ref):
    x = x_ref[...]
    h = jnp.dot(x, win_ref[...], preferred_element_type=jnp.float32)
    m = jnp.mean(h, axis=0, keepdims=True)
    v = jnp.mean((h - m) ** 2, axis=0, keepdims=True)
    h = (h - m) / jnp.sqrt(v + EPS) * g1_ref[...] + b1_ref[...]
    h = jnp.maximum(h, 0.0)
    asrc_ref[...] = jnp.dot(h, wsrc_ref[...], preferred_element_type=jnp.float32) + bsrc_ref[...]
    adst_ref[...] = jnp.dot(h, wdst_ref[...], preferred_element_type=jnp.float32) + bdst_ref[...]
    xl_ref[...] = jnp.dot(h, wlin_ref[...], preferred_element_type=jnp.float32) + blin_ref[...]


def _post_body(out_ref, g2_ref, b2_ref, wout_ref, g3_ref, b3_ref, xskip_ref, y_ref):
    o = out_ref[...]
    m = jnp.mean(o, axis=0, keepdims=True)
    v = jnp.mean((o - m) ** 2, axis=0, keepdims=True)
    h = (o - m) / jnp.sqrt(v + EPS) * g2_ref[...] + b2_ref[...]
    h = jnp.maximum(h, 0.0)
    h = jnp.dot(h, wout_ref[...], preferred_element_type=jnp.float32)
    m = jnp.mean(h, axis=0, keepdims=True)
    v = jnp.mean((h - m) ** 2, axis=0, keepdims=True)
    h = (h - m) / jnp.sqrt(v + EPS) * g3_ref[...] + b3_ref[...]
    y_ref[...] = jnp.maximum(h + xskip_ref[...], 0.0)


def kernel(x, pos, edge_index, params):
    p = params
    src, dst = edge_index[0], edge_index[1]

    r2 = lambda a: a.reshape(1, -1)
    asrc, adst, xl = pl.pallas_call(
        _pre_body,
        out_shape=[jax.ShapeDtypeStruct((N, C), jnp.float32)] * 3,
    )(x, p['lin_in_W'], r2(p['bn1_g']), r2(p['bn1_b']),
      p['pt_src_W'], r2(p['pt_src_b']), p['pt_dst_W'], r2(p['pt_dst_b']),
      p['pt_lin_W'], r2(p['pt_lin_b']))

    a_j = asrc[src]
    a_i = adst[dst]
    rel = pos[src] - pos[dst]
    q = rel @ p['pos_W1'] + p['pos_b1']
    m = q.mean(0)
    v = ((q - m) ** 2).mean(0)
    d = jnp.maximum((q - m) / jnp.sqrt(v + EPS) * p['pos_bn_g'] + p['pos_bn_b'], 0.0)
    delta = d @ p['pos_W2'] + p['pos_b2']
    a = a_j - a_i + delta
    m = a.mean(0)
    v = ((a - m) ** 2).mean(0)
    a = jnp.maximum((a - m) / jnp.sqrt(v + EPS) * p['attn_bn1_g'] + p['attn_bn1_b'], 0.0)
    t = a @ p['attn_W1'] + p['attn_b1']
    m = t.mean(0)
    v = ((t - m) ** 2).mean(0)
    t = jnp.maximum((t - m) / jnp.sqrt(v + EPS) * p['attn_bn2_g'] + p['attn_bn2_b'], 0.0)
    af = t @ p['attn_W2'] + p['attn_b2']
    ae = jnp.exp(af)
    asum = jax.ops.segment_sum(ae, dst, num_segments=N)
    alpha = ae / (asum[dst] + 1e-16)
    xj = xl[src]
    msg = (alpha[:, None, :] * (xj + delta).reshape(E, SP, CA)).reshape(E, C)
    out = jax.ops.segment_sum(msg, dst, num_segments=N)

    y = pl.pallas_call(
        _post_body,
        out_shape=jax.ShapeDtypeStruct((N, C), jnp.float32),
    )(out, r2(p['bn2_g']), r2(p['bn2_b']), p['lin_out_W'],
      r2(p['bn3_g']), r2(p['bn3_b']), x)
    return y


# speedup vs baseline: 3.5026x; 2.8374x over previous
import functools
import jax
import jax.numpy as jnp
from jax import lax
from jax.experimental import pallas as pl
from jax.experimental.pallas import tpu as pltpu
from jax.experimental.pallas import tpu_sc as plsc

N = 10000
E = 320000
C = 128
SP = 8
CA = C // SP
EPS = 1e-5

NC = 2
NS = 16
NW = NC * NS
EW = E // NW
G = 80
NCHUNK = EW // G
NGRP = G // 16
NPAD = 10240
NPAD8 = NPAD // 8


def _row16(ref, row, col0):
    return ref[row, pl.ds(col0, 16)]


def _mesh():
    return plsc.VectorSubcoreMesh(core_axis_name="c", subcore_axis_name="s")


_SC_PARAMS = pltpu.CompilerParams(needs_layout_passes=False)


def _sca_body(pos_hbm, src_hbm, dst_hbm, w_hbm, stat_hbm,
              pos_v, sbuf, dbuf, wv, statv, sem):
    c = lax.axis_index("c")
    s = lax.axis_index("s")
    wid = c * NS + s
    base = wid * EW
    pltpu.sync_copy(pos_hbm, pos_v)
    pltpu.sync_copy(w_hbm, wv)
    wvec = wv[pl.ds(0, 16)]
    wsc = [wvec[i] for i in range(12)]

    def chunk(i, acc):
        cb = base + i * G
        pltpu.sync_copy(src_hbm.at[pl.ds(cb, G)], sbuf)
        pltpu.sync_copy(dst_hbm.at[pl.ds(cb, G)], dbuf)

        def grp(g, acc):
            si = sbuf[pl.ds(g * 16, 16)]
            di = dbuf[pl.ds(g * 16, 16)]
            ps = [plsc.load_gather(pos_v, [si * 3 + k]) for k in range(3)]
            pd = [plsc.load_gather(pos_v, [di * 3 + k]) for k in range(3)]
            rel = [ps[k] - pd[k] for k in range(3)]
            q0 = rel[0] * wsc[0] + rel[1] * wsc[1] + rel[2] * wsc[2] + wsc[3]
            q1 = rel[0] * wsc[4] + rel[1] * wsc[5] + rel[2] * wsc[6] + wsc[7]
            q2 = rel[0] * wsc[8] + rel[1] * wsc[9] + rel[2] * wsc[10] + wsc[11]
            return (acc[0] + q0, acc[1] + q1, acc[2] + q2,
                    acc[3] + q0 * q0, acc[4] + q1 * q1, acc[5] + q2 * q2)

        return lax.fori_loop(0, NGRP, grp, acc)

    z = jnp.zeros((16,), jnp.float32)
    acc = lax.fori_loop(0, NCHUNK, chunk, (z, z, z, z, z, z))
    for k in range(6):
        statv[0, pl.ds(16 * k, 16)] = acc[k]
    statv[0, pl.ds(96, 16)] = z
    statv[0, pl.ds(112, 16)] = z
    pltpu.sync_copy(statv, stat_hbm.at[wid])


def _sc_a(pos_flat, src, dst, w1pack):
    kfn = pl.kernel(
        _sca_body,
        out_type=jax.ShapeDtypeStruct((NW, 1, 128), jnp.float32),
        mesh=_mesh(),
        compiler_params=_SC_PARAMS,
        scratch_types=[
            pltpu.VMEM((3 * N,), jnp.float32),
            pltpu.VMEM((G,), jnp.int32),
            pltpu.VMEM((G,), jnp.int32),
            pltpu.VMEM((16,), jnp.float32),
            pltpu.VMEM((1, 128), jnp.float32),
            pltpu.SemaphoreType.DMA,
        ],
    )
    return kfn(pos_flat, src, dst, w1pack)


def _scb_body(asrc_hbm, adst_hbm, xl_hbm, pos_hbm, src_hbm, dst_hbm,
              pf_hbm, w2_hbm,
              a_hbm, xpd_hbm, stat_hbm,
              pos_v, sbuf, dbuf, pfv, w2v, ajb, aib, xjb, ab, statv, sem):
    c = lax.axis_index("c")
    s = lax.axis_index("s")
    wid = c * NS + s
    base = wid * EW
    pltpu.sync_copy(pos_hbm, pos_v)
    pltpu.sync_copy(pf_hbm, pfv)
    pltpu.sync_copy(w2_hbm, w2v)
    pfa = pfv[0, pl.ds(0, 16)]
    pfb = pfv[0, pl.ds(16, 16)]
    pf = lambda j: pfa[j] if j < 16 else pfb[j - 16]
    sc0, sc1, sc2 = pf(0), pf(1), pf(2)
    sh0, sh1, sh2 = pf(3), pf(4), pf(5)
    w2w = [[w2v[k, pl.ds(16 * v, 16)] for v in range(SP)] for k in range(3)]
    bw = [w2v[3, pl.ds(16 * v, 16)] for v in range(SP)]

    def chunk(i, acc):
        cb = base + i * G
        pltpu.sync_copy(src_hbm.at[pl.ds(cb, G)], sbuf)
        pltpu.sync_copy(dst_hbm.at[pl.ds(cb, G)], dbuf)
        pltpu.async_copy(asrc_hbm.at[sbuf], ajb, sem).wait()
        pltpu.async_copy(adst_hbm.at[dbuf], aib, sem).wait()
        pltpu.async_copy(xl_hbm.at[sbuf], xjb, sem).wait()

        def grp(g, acc):
            accs, accq = acc
            si = sbuf[pl.ds(g * 16, 16)]
            di = dbuf[pl.ds(g * 16, 16)]
            ps = [plsc.load_gather(pos_v, [si * 3 + k]) for k in range(3)]
            pd = [plsc.load_gather(pos_v, [di * 3 + k]) for k in range(3)]
            rel = [ps[k] - pd[k] for k in range(3)]
            d0 = jnp.maximum(
                (rel[0] * pf(6) + rel[1] * pf(7) + rel[2] * pf(8)
                 + pf(9)) * sc0 + sh0, 0.0)
            d1 = jnp.maximum(
                (rel[0] * pf(10) + rel[1] * pf(11) + rel[2] * pf(12)
                 + pf(13)) * sc1 + sh1, 0.0)
            d2 = jnp.maximum(
                (rel[0] * pf(14) + rel[1] * pf(15) + rel[2] * pf(16)
                 + pf(17)) * sc2 + sh2, 0.0)
            naccs, naccq = list(accs), list(accq)
            for e in range(16):
                row = g * 16 + e
                e0, e1, e2 = d0[e], d1[e], d2[e]
                for v in range(SP):
                    delta = (bw[v] + e0 * w2w[0][v] + e1 * w2w[1][v]
                             + e2 * w2w[2][v])
                    aj = _row16(ajb, row, 16 * v)
                    ai = _row16(aib, row, 16 * v)
                    a = aj - ai + delta
                    naccs[v] = naccs[v] + a
                    naccq[v] = naccq[v] + a * a
                    ab[row, pl.ds(16 * v, 16)] = a
                    xj = _row16(xjb, row, 16 * v)
                    xjb[row, pl.ds(16 * v, 16)] = xj + delta
            return (tuple(naccs), tuple(naccq))

        acc = lax.fori_loop(0, NGRP, grp, acc)
        pltpu.sync_copy(ab, a_hbm.at[pl.ds(cb, G)])
        pltpu.sync_copy(xjb, xpd_hbm.at[pl.ds(cb, G)])
        return acc

    z = jnp.zeros((16,), jnp.float32)
    acc = lax.fori_loop(0, NCHUNK, chunk,
                        ((z,) * SP, (z,) * SP))
    for v in range(SP):
        statv[0, pl.ds(16 * v, 16)] = acc[0][v]
        statv[1, pl.ds(16 * v, 16)] = acc[1][v]
    pltpu.sync_copy(statv, stat_hbm.at[wid])


def _sc_b(asrc, adst, xl, pos_flat, src, dst, pfpack, w2pack):
    kfn = pl.kernel(
        _scb_body,
        out_type=[jax.ShapeDtypeStruct((E, C), jnp.float32),
                  jax.ShapeDtypeStruct((E, C), jnp.float32),
                  jax.ShapeDtypeStruct((NW, 2, C), jnp.float32)],
        mesh=_mesh(),
        compiler_params=_SC_PARAMS,
        scratch_types=[
            pltpu.VMEM((3 * N,), jnp.float32),
            pltpu.VMEM((G,), jnp.int32),
            pltpu.VMEM((G,), jnp.int32),
            pltpu.VMEM((1, 128), jnp.float32),
            pltpu.VMEM((4, C), jnp.float32),
            pltpu.VMEM((G, C), jnp.float32),
            pltpu.VMEM((G, C), jnp.float32),
            pltpu.VMEM((G, C), jnp.float32),
            pltpu.VMEM((G, C), jnp.float32),
            pltpu.VMEM((2, C), jnp.float32),
            pltpu.SemaphoreType.DMA,
        ],
    )
    return kfn(asrc, adst, xl, pos_flat, src, dst, pfpack, w2pack)


def _scc_body(ae_hbm, xpd_hbm, dst_hbm, outp_hbm, aesum_hbm,
              dbuf, d8buf, aeidx, aeb, aw, msgb, zb,
              shared, shared2, sem):
    c = lax.axis_index("c")
    s = lax.axis_index("s")
    wid = c * NS + s
    base = wid * EW
    rows = NPAD // NS
    rows2 = NPAD8 // NS

    def zrow(i, carry):
        for v in range(SP):
            zb[i, pl.ds(16 * v, 16)] = jnp.zeros((16,), jnp.float32)
        return carry

    lax.fori_loop(0, 80, zrow, 0)
    for z8 in range(8):
        pltpu.sync_copy(zb, shared.at[pl.ds(s * rows + z8 * 80, 80)])
    pltpu.sync_copy(zb, shared2.at[pl.ds(s * 80, 80)])
    plsc.subcore_barrier()

    iota16 = jnp.minimum(lax.iota(jnp.int32, 16), G // 8 - 1)

    def chunk(i, carry):
        cb = base + i * G
        pltpu.sync_copy(dst_hbm.at[pl.ds(cb, G)], dbuf)
        aeidx[...] = iota16 + lax.div(cb, 8)
        pltpu.async_copy(ae_hbm.at[aeidx], aeb, sem).wait()
        pltpu.sync_copy(xpd_hbm.at[pl.ds(cb, G)], msgb)

        def grp(g, carry2):
            di = dbuf[pl.ds(g * 16, 16)]
            d8buf[pl.ds(g * 16, 16)] = lax.shift_right_logical(di, 3)
            col16 = lax.shift_left(jnp.bitwise_and(di, 7), 4)
            for e in range(16):
                row = g * 16 + e
                ae = aeb[g * 2 + e // 8, pl.ds((e % 8) * 16, 16)]
                for v in range(SP):
                    aw[row, pl.ds(16 * v, 16)] = jnp.zeros((16,), jnp.float32)
                for v in range(SP):
                    msgb[row, pl.ds(16 * v, 16)] = (
                        ae * _row16(msgb, row, 16 * v))
                aw[row, pl.ds(col16[e], 16)] = ae
            return carry2

        lax.fori_loop(0, NGRP, grp, 0)
        pltpu.sync_copy(msgb, shared.at[dbuf], add=True)
        pltpu.sync_copy(aw, shared2.at[d8buf], add=True)
        return carry

    lax.fori_loop(0, NCHUNK, chunk, 0)
    plsc.subcore_barrier()
    pltpu.sync_copy(shared.at[pl.ds(s * rows, rows)],
                    outp_hbm.at[pl.ds(c * NPAD + s * rows, rows)])
    pltpu.sync_copy(shared2.at[pl.ds(s * rows2, rows2)],
                    aesum_hbm.at[pl.ds(c * NPAD8 + s * rows2, rows2)])


def _sc_c(ae_pk, xpd, dst):
    kfn = pl.kernel(
        _scc_body,
        out_type=[jax.ShapeDtypeStruct((NC * NPAD, C), jnp.float32),
                  jax.ShapeDtypeStruct((NC * NPAD8, C), jnp.float32)],
        mesh=_mesh(),
        compiler_params=_SC_PARAMS,
        scratch_types=[
            pltpu.VMEM((G,), jnp.int32),
            pltpu.VMEM((G,), jnp.int32),
            pltpu.VMEM((16,), jnp.int32),
            pltpu.VMEM((16, C), jnp.float32),
            pltpu.VMEM((G, C), jnp.float32),
            pltpu.VMEM((G, C), jnp.float32),
            pltpu.VMEM((80, C), jnp.float32),
            pltpu.VMEM_SHARED((NPAD, C), jnp.float32),
            pltpu.VMEM_SHARED((NPAD8, C), jnp.float32),
            pltpu.SemaphoreType.DMA,
        ],
    )
    return kfn(ae_pk, xpd, dst)


def _pre_body(x_ref, win_ref, g1_ref, b1_ref, wsrc_ref, bsrc_ref,
              wdst_ref, bdst_ref, wlin_ref, blin_ref,
              qstat_ref, posg_ref, posb_ref,
              asrc_ref, adst_ref, xl_ref, pf_ref):
    x = x_ref[...]
    h = jnp.dot(x, win_ref[...], preferred_element_type=jnp.float32)
    m = jnp.mean(h, axis=0, keepdims=True)
    v = jnp.mean((h - m) ** 2, axis=0, keepdims=True)
    h = (h - m) / jnp.sqrt(v + EPS) * g1_ref[...] + b1_ref[...]
    h = jnp.maximum(h, 0.0)
    asrc_ref[...] = jnp.dot(h, wsrc_ref[...], preferred_element_type=jnp.float32) + bsrc_ref[...]
    adst_ref[...] = jnp.dot(h, wdst_ref[...], preferred_element_type=jnp.float32) + bdst_ref[...]
    xl_ref[...] = jnp.dot(h, wlin_ref[...], preferred_element_type=jnp.float32) + blin_ref[...]
    st = jnp.sum(qstat_ref[...], axis=(0, 1)).reshape(1, 128)
    lane = lax.broadcasted_iota(jnp.int32, (1, 128), 1)
    grpid = lane // 16
    sums = [jnp.sum(jnp.where(grpid == k, st, 0.0)) for k in range(6)]
    out_row = jnp.zeros((1, 128), jnp.float32)
    for k in range(3):
        mq = sums[k] / E
        vq = sums[k + 3] / E - mq * mq
        scale = posg_ref[0, k] / jnp.sqrt(vq + EPS)
        shift = posb_ref[0, k] - mq * scale
        out_row = jnp.where(lane == k, scale, out_row)
        out_row = jnp.where(lane == 3 + k, shift, out_row)
    pf_ref[...] = out_row


def _t3a_body(a_ref, bstat_ref, g1_ref, b1_ref, w1_ref, bb1_ref,
              t_ref, ts_ref, tq_ref):
    i = pl.program_id(0)
    st = jnp.sum(bstat_ref[...], axis=0)
    m = st[0:1] / E
    v = st[1:2] / E - m * m
    scale = g1_ref[...] / jnp.sqrt(v + EPS)
    shift = b1_ref[...] - m * scale
    a = jnp.maximum(a_ref[...] * scale + shift, 0.0)
    t = jnp.dot(a, w1_ref[...], preferred_element_type=jnp.float32) + bb1_ref[...]
    t_ref[...] = t

    @pl.when(i == 0)
    def _():
        ts_ref[...] = jnp.zeros_like(ts_ref)
        tq_ref[...] = jnp.zeros_like(tq_ref)

    ts_ref[...] += jnp.sum(t, axis=0, keepdims=True)
    tq_ref[...] += jnp.sum(t * t, axis=0, keepdims=True)


def _t3b_body(t_ref, ts_ref, tq_ref, g2_ref, b2_ref, w2_ref, bb2_ref, ae_ref):
    m = ts_ref[...] / E
    v = tq_ref[...] / E - m * m
    scale = g2_ref[...] / jnp.sqrt(v + EPS)
    shift = b2_ref[...] - m * scale
    t = jnp.maximum(t_ref[...] * scale + shift, 0.0)
    af = jnp.dot(t, w2_ref[...], preferred_element_type=jnp.float32) + bb2_ref[...]
    ae_ref[...] = jnp.exp(af)


def _norm_body(outp_ref, asum_ref, o_ref):
    denom = asum_ref[0] + asum_ref[1] + 1e-16
    denom = jnp.concatenate([denom] * SP, axis=1)
    o_ref[...] = (outp_ref[0] + outp_ref[1]) / denom


def _post_body(o_ref, g2_ref, b2_ref, wout_ref, g3_ref, b3_ref,
               xskip_ref, y_ref):
    o = o_ref[...]
    m = jnp.mean(o, axis=0, keepdims=True)
    v = jnp.mean((o - m) ** 2, axis=0, keepdims=True)
    h = (o - m) / jnp.sqrt(v + EPS) * g2_ref[...] + b2_ref[...]
    h = jnp.maximum(h, 0.0)
    h = jnp.dot(h, wout_ref[...], preferred_element_type=jnp.float32)
    m = jnp.mean(h, axis=0, keepdims=True)
    v = jnp.mean((h - m) ** 2, axis=0, keepdims=True)
    h = (h - m) / jnp.sqrt(v + EPS) * g3_ref[...] + b3_ref[...]
    y_ref[...] = jnp.maximum(h + xskip_ref[...], 0.0)


def kernel(x, pos, edge_index, params):
    p = params
    src, dst = edge_index[0], edge_index[1]
    pos_flat = pos.reshape(-1)
    r2 = lambda a: a.reshape(1, -1)

    w1pack = jnp.pad(
        jnp.concatenate([p['pos_W1'], p['pos_b1'][None, :]], axis=0).T.reshape(-1),
        (0, 4))
    w2pack = jnp.concatenate([p['pos_W2'], p['pos_b2'][None, :]], axis=0)

    qstat = _sc_a(pos_flat, src, dst, w1pack)

    asrc, adst, xl, pf0 = pl.pallas_call(
        _pre_body,
        out_shape=[jax.ShapeDtypeStruct((N, C), jnp.float32)] * 3
        + [jax.ShapeDtypeStruct((1, 128), jnp.float32)],
    )(x, p['lin_in_W'], r2(p['bn1_g']), r2(p['bn1_b']),
      p['pt_src_W'], r2(p['pt_src_b']), p['pt_dst_W'], r2(p['pt_dst_b']),
      p['pt_lin_W'], r2(p['pt_lin_b']),
      qstat, r2(jnp.pad(p['pos_bn_g'], (0, 13))), r2(jnp.pad(p['pos_bn_b'], (0, 13))))

    pfpack = jnp.concatenate([pf0[:, 0:6], w1pack[None, 0:12],
                              jnp.zeros((1, 110), jnp.float32)], axis=1)

    a, xpd, bstat = _sc_b(asrc, adst, xl, pos_flat, src, dst, pfpack, w2pack)

    BE = 5000
    nb = E // BE
    t, ts, tq = pl.pallas_call(
        _t3a_body,
        grid=(nb,),
        in_specs=[
            pl.BlockSpec((BE, C), lambda i: (i, 0)),
            pl.BlockSpec((NW, 2, C), lambda i: (0, 0, 0)),
            pl.BlockSpec((1, C), lambda i: (0, 0)),
            pl.BlockSpec((1, C), lambda i: (0, 0)),
            pl.BlockSpec((C, CA), lambda i: (0, 0)),
            pl.BlockSpec((1, CA), lambda i: (0, 0)),
        ],
        out_specs=[
            pl.BlockSpec((BE, CA), lambda i: (i, 0)),
            pl.BlockSpec((1, CA), lambda i: (0, 0)),
            pl.BlockSpec((1, CA), lambda i: (0, 0)),
        ],
        out_shape=[jax.ShapeDtypeStruct((E, CA), jnp.float32),
                   jax.ShapeDtypeStruct((1, CA), jnp.float32),
                   jax.ShapeDtypeStruct((1, CA), jnp.float32)],
    )(a, bstat, r2(p['attn_bn1_g']), r2(p['attn_bn1_b']),
      p['attn_W1'], r2(p['attn_b1']))

    ae = pl.pallas_call(
        _t3b_body,
        grid=(nb,),
        in_specs=[
            pl.BlockSpec((BE, CA), lambda i: (i, 0)),
            pl.BlockSpec((1, CA), lambda i: (0, 0)),
            pl.BlockSpec((1, CA), lambda i: (0, 0)),
            pl.BlockSpec((1, CA), lambda i: (0, 0)),
            pl.BlockSpec((1, CA), lambda i: (0, 0)),
            pl.BlockSpec((CA, CA), lambda i: (0, 0)),
            pl.BlockSpec((1, CA), lambda i: (0, 0)),
        ],
        out_specs=pl.BlockSpec((BE, CA), lambda i: (i, 0)),
        out_shape=jax.ShapeDtypeStruct((E, CA), jnp.float32),
    )(t, ts, tq, r2(p['attn_bn2_g']), r2(p['attn_bn2_b']),
      p['attn_W2'], r2(p['attn_b2']))

    ae_pk = ae.reshape(E // 8, C)
    outp, aesum = _sc_c(ae_pk, xpd, dst)
    outp = outp.reshape(NC, NPAD, C)[:, :N, :]
    asum = aesum.reshape(NC, NPAD, CA)[:, :N, :]

    BN = 2000
    o = pl.pallas_call(
        _norm_body,
        grid=(N // BN,),
        in_specs=[
            pl.BlockSpec((NC, BN, C), lambda i: (0, i, 0)),
            pl.BlockSpec((NC, BN, CA), lambda i: (0, i, 0)),
        ],
        out_specs=pl.BlockSpec((BN, C), lambda i: (i, 0)),
        out_shape=jax.ShapeDtypeStruct((N, C), jnp.float32),
    )(outp, asum)

    y = pl.pallas_call(
        _post_body,
        out_shape=jax.ShapeDtypeStruct((N, C), jnp.float32),
    )(o, r2(p['bn2_g']), r2(p['bn2_b']), p['lin_out_W'],
      r2(p['bn3_g']), r2(p['bn3_b']), x)
    return y


# speedup vs baseline: 3.9380x; 1.1243x over previous
import functools
import jax
import jax.numpy as jnp
from jax import lax
from jax.experimental import pallas as pl
from jax.experimental.pallas import tpu as pltpu
from jax.experimental.pallas import tpu_sc as plsc

N = 10000
E = 320000
C = 128
SP = 8
CA = C // SP
EPS = 1e-5

NC = 2
NS = 16
NW = NC * NS
EW = E // NW
G = 80
NCHUNK = EW // G
NGRP = G // 16
NPAD = 10240
NPAD8 = NPAD // 8


def _row16(ref, row, col0):
    return ref[row, pl.ds(col0, 16)]


def _mesh():
    return plsc.VectorSubcoreMesh(core_axis_name="c", subcore_axis_name="s")


_SC_PARAMS = pltpu.CompilerParams(needs_layout_passes=False)


def _sca_body(pos_hbm, src_hbm, dst_hbm, w_hbm, stat_hbm,
              pos_v, sbuf, dbuf, wv, statv, sem):
    c = lax.axis_index("c")
    s = lax.axis_index("s")
    wid = c * NS + s
    base = wid * EW
    pltpu.sync_copy(pos_hbm, pos_v)
    pltpu.sync_copy(w_hbm, wv)
    wvec = wv[pl.ds(0, 16)]
    wsc = [wvec[i] for i in range(12)]

    def chunk(i, acc):
        cb = base + i * G
        pltpu.sync_copy(src_hbm.at[pl.ds(cb, G)], sbuf)
        pltpu.sync_copy(dst_hbm.at[pl.ds(cb, G)], dbuf)

        def grp(g, acc):
            si = sbuf[pl.ds(g * 16, 16)]
            di = dbuf[pl.ds(g * 16, 16)]
            ps = [plsc.load_gather(pos_v, [si * 3 + k]) for k in range(3)]
            pd = [plsc.load_gather(pos_v, [di * 3 + k]) for k in range(3)]
            rel = [ps[k] - pd[k] for k in range(3)]
            q0 = rel[0] * wsc[0] + rel[1] * wsc[1] + rel[2] * wsc[2] + wsc[3]
            q1 = rel[0] * wsc[4] + rel[1] * wsc[5] + rel[2] * wsc[6] + wsc[7]
            q2 = rel[0] * wsc[8] + rel[1] * wsc[9] + rel[2] * wsc[10] + wsc[11]
            return (acc[0] + q0, acc[1] + q1, acc[2] + q2,
                    acc[3] + q0 * q0, acc[4] + q1 * q1, acc[5] + q2 * q2)

        return lax.fori_loop(0, NGRP, grp, acc)

    z = jnp.zeros((16,), jnp.float32)
    acc = lax.fori_loop(0, NCHUNK, chunk, (z, z, z, z, z, z))
    for k in range(6):
        statv[0, pl.ds(16 * k, 16)] = acc[k]
    statv[0, pl.ds(96, 16)] = z
    statv[0, pl.ds(112, 16)] = z
    pltpu.sync_copy(statv, stat_hbm.at[wid])


def _sc_a(pos_flat, src, dst, w1pack):
    kfn = pl.kernel(
        _sca_body,
        out_type=jax.ShapeDtypeStruct((NW, 1, 128), jnp.float32),
        mesh=_mesh(),
        compiler_params=_SC_PARAMS,
        scratch_types=[
            pltpu.VMEM((3 * N,), jnp.float32),
            pltpu.VMEM((G,), jnp.int32),
            pltpu.VMEM((G,), jnp.int32),
            pltpu.VMEM((16,), jnp.float32),
            pltpu.VMEM((1, 128), jnp.float32),
            pltpu.SemaphoreType.DMA,
        ],
    )
    return kfn(pos_flat, src, dst, w1pack)


def _scb_body(asrc_hbm, adst_hbm, xl_hbm, pos_hbm, src_hbm, dst_hbm,
              pf_hbm, w2_hbm,
              a_hbm, xpd_hbm, stat_hbm,
              pos_v, sbuf0, dbuf0, sbuf1, dbuf1, pfv, w2v,
              ajb0, aib0, xjb0, ab0, ajb1, aib1, xjb1, ab1, statv,
              gsem, wsem):
    c = lax.axis_index("c")
    s = lax.axis_index("s")
    wid = c * NS + s
    base = wid * EW
    pltpu.sync_copy(pos_hbm, pos_v)
    pltpu.sync_copy(pf_hbm, pfv)
    pltpu.sync_copy(w2_hbm, w2v)
    pfa = pfv[0, pl.ds(0, 16)]
    pfb = pfv[0, pl.ds(16, 16)]
    pf = lambda j: pfa[j] if j < 16 else pfb[j - 16]
    sc0, sc1, sc2 = pf(0), pf(1), pf(2)
    sh0, sh1, sh2 = pf(3), pf(4), pf(5)
    w2w = [[w2v[k, pl.ds(16 * v, 16)] for v in range(SP)] for k in range(3)]
    bw = [w2v[3, pl.ds(16 * v, 16)] for v in range(SP)]

    SB = (sbuf0, sbuf1)
    DB = (dbuf0, dbuf1)
    AJ = (ajb0, ajb1)
    AI = (aib0, aib1)
    XJ = (xjb0, xjb1)
    AB = (ab0, ab1)

    def load_idx(cb, b):
        pltpu.sync_copy(src_hbm.at[pl.ds(cb, G)], SB[b])
        pltpu.sync_copy(dst_hbm.at[pl.ds(cb, G)], DB[b])

    def start_gathers(b):
        pltpu.async_copy(asrc_hbm.at[SB[b]], AJ[b], gsem)
        pltpu.async_copy(adst_hbm.at[DB[b]], AI[b], gsem)
        pltpu.async_copy(xl_hbm.at[SB[b]], XJ[b], gsem)

    def wait_gathers(b):
        pltpu.make_async_copy(asrc_hbm.at[SB[b]], AJ[b], gsem).wait()
        pltpu.make_async_copy(adst_hbm.at[DB[b]], AI[b], gsem).wait()
        pltpu.make_async_copy(xl_hbm.at[SB[b]], XJ[b], gsem).wait()

    def start_writes(cb, b):
        pltpu.async_copy(AB[b], a_hbm.at[pl.ds(cb, G)], wsem)
        pltpu.async_copy(XJ[b], xpd_hbm.at[pl.ds(cb, G)], wsem)

    def wait_writes(cb, b):
        pltpu.make_async_copy(AB[b], a_hbm.at[pl.ds(cb, G)], wsem).wait()
        pltpu.make_async_copy(XJ[b], xpd_hbm.at[pl.ds(cb, G)], wsem).wait()

    def compute(b, acc):
        ajb, aib, xjb, ab = AJ[b], AI[b], XJ[b], AB[b]
        sbuf, dbuf = SB[b], DB[b]

        def grp(g, acc):
            accs, accq = acc
            si = sbuf[pl.ds(g * 16, 16)]
            di = dbuf[pl.ds(g * 16, 16)]
            ps = [plsc.load_gather(pos_v, [si * 3 + k]) for k in range(3)]
            pd = [plsc.load_gather(pos_v, [di * 3 + k]) for k in range(3)]
            rel = [ps[k] - pd[k] for k in range(3)]
            d0 = jnp.maximum(
                (rel[0] * pf(6) + rel[1] * pf(7) + rel[2] * pf(8)
                 + pf(9)) * sc0 + sh0, 0.0)
            d1 = jnp.maximum(
                (rel[0] * pf(10) + rel[1] * pf(11) + rel[2] * pf(12)
                 + pf(13)) * sc1 + sh1, 0.0)
            d2 = jnp.maximum(
                (rel[0] * pf(14) + rel[1] * pf(15) + rel[2] * pf(16)
                 + pf(17)) * sc2 + sh2, 0.0)
            naccs, naccq = list(accs), list(accq)
            for e in range(16):
                row = g * 16 + e
                e0, e1, e2 = d0[e], d1[e], d2[e]
                for v in range(SP):
                    delta = (bw[v] + e0 * w2w[0][v] + e1 * w2w[1][v]
                             + e2 * w2w[2][v])
                    aj = _row16(ajb, row, 16 * v)
                    ai = _row16(aib, row, 16 * v)
                    a = aj - ai + delta
                    naccs[v] = naccs[v] + a
                    naccq[v] = naccq[v] + a * a
                    ab[row, pl.ds(16 * v, 16)] = a
                    xj = _row16(xjb, row, 16 * v)
                    xjb[row, pl.ds(16 * v, 16)] = xj + delta
            return (tuple(naccs), tuple(naccq))

        return lax.fori_loop(0, NGRP, grp, acc)

    def pair(j, acc):
        c0 = base + (2 * j) * G
        c1 = c0 + G
        c2 = c1 + G
        wait_gathers(0)

        @pl.when(j > 0)
        def _():
            wait_writes(c0 - G, 1)

        load_idx(c1, 1)
        start_gathers(1)
        acc = compute(0, acc)
        start_writes(c0, 0)
        wait_gathers(1)
        wait_writes(c0, 0)
        load_idx(c2, 0)
        start_gathers(0)
        acc = compute(1, acc)
        start_writes(c1, 1)
        return acc

    z = jnp.zeros((16,), jnp.float32)
    load_idx(base, 0)
    start_gathers(0)
    acc = lax.fori_loop(0, (NCHUNK - 1) // 2, pair, ((z,) * SP, (z,) * SP))
    clast = base + (NCHUNK - 1) * G
    wait_gathers(0)
    wait_writes(clast - G, 1)
    acc = compute(0, acc)
    start_writes(clast, 0)
    wait_writes(clast, 0)
    for v in range(SP):
        statv[0, pl.ds(16 * v, 16)] = acc[0][v]
        statv[1, pl.ds(16 * v, 16)] = acc[1][v]
    pltpu.sync_copy(statv, stat_hbm.at[wid])


def _sc_b(asrc, adst, xl, pos_flat, src, dst, pfpack, w2pack):
    kfn = pl.kernel(
        _scb_body,
        out_type=[jax.ShapeDtypeStruct((E, C), jnp.float32),
                  jax.ShapeDtypeStruct((E, C), jnp.float32),
                  jax.ShapeDtypeStruct((NW, 2, C), jnp.float32)],
        mesh=_mesh(),
        compiler_params=_SC_PARAMS,
        scratch_types=[
            pltpu.VMEM((3 * N,), jnp.float32),
            pltpu.VMEM((G,), jnp.int32),
            pltpu.VMEM((G,), jnp.int32),
            pltpu.VMEM((G,), jnp.int32),
            pltpu.VMEM((G,), jnp.int32),
            pltpu.VMEM((1, 128), jnp.float32),
            pltpu.VMEM((4, C), jnp.float32),
            pltpu.VMEM((G, C), jnp.float32),
            pltpu.VMEM((G, C), jnp.float32),
            pltpu.VMEM((G, C), jnp.float32),
            pltpu.VMEM((G, C), jnp.float32),
            pltpu.VMEM((G, C), jnp.float32),
            pltpu.VMEM((G, C), jnp.float32),
            pltpu.VMEM((G, C), jnp.float32),
            pltpu.VMEM((G, C), jnp.float32),
            pltpu.VMEM((2, C), jnp.float32),
            pltpu.SemaphoreType.DMA,
            pltpu.SemaphoreType.DMA,
        ],
    )
    return kfn(asrc, adst, xl, pos_flat, src, dst, pfpack, w2pack)


def _scc_body(ae_hbm, xpd_hbm, dst_hbm, outp_hbm, aesum_hbm,
              dbuf, d8buf, aeidx, aeb, aw, msgb, zb,
              shared, shared2, sem):
    c = lax.axis_index("c")
    s = lax.axis_index("s")
    wid = c * NS + s
    base = wid * EW
    rows = NPAD // NS
    rows2 = NPAD8 // NS

    def zrow(i, carry):
        for v in range(SP):
            zb[i, pl.ds(16 * v, 16)] = jnp.zeros((16,), jnp.float32)
        return carry

    lax.fori_loop(0, 80, zrow, 0)
    for z8 in range(8):
        pltpu.sync_copy(zb, shared.at[pl.ds(s * rows + z8 * 80, 80)])
    pltpu.sync_copy(zb, shared2.at[pl.ds(s * 80, 80)])
    plsc.subcore_barrier()

    iota16 = jnp.minimum(lax.iota(jnp.int32, 16), G // 8 - 1)

    def chunk(i, carry):
        cb = base + i * G
        pltpu.sync_copy(dst_hbm.at[pl.ds(cb, G)], dbuf)
        aeidx[...] = iota16 + lax.div(cb, 8)
        pltpu.async_copy(ae_hbm.at[aeidx], aeb, sem)
        pltpu.async_copy(xpd_hbm.at[pl.ds(cb, G)], msgb, sem)
        pltpu.make_async_copy(ae_hbm.at[aeidx], aeb, sem).wait()
        pltpu.make_async_copy(xpd_hbm.at[pl.ds(cb, G)], msgb, sem).wait()

        def grp(g, carry2):
            di = dbuf[pl.ds(g * 16, 16)]
            d8buf[pl.ds(g * 16, 16)] = lax.shift_right_logical(di, 3)
            col16 = lax.shift_left(jnp.bitwise_and(di, 7), 4)
            for e in range(16):
                row = g * 16 + e
                ae = aeb[g * 2 + e // 8, pl.ds((e % 8) * 16, 16)]
                for v in range(SP):
                    aw[row, pl.ds(16 * v, 16)] = jnp.zeros((16,), jnp.float32)
                for v in range(SP):
                    msgb[row, pl.ds(16 * v, 16)] = (
                        ae * _row16(msgb, row, 16 * v))
                aw[row, pl.ds(col16[e], 16)] = ae
            return carry2

        lax.fori_loop(0, NGRP, grp, 0)
        pltpu.async_copy(msgb, shared.at[dbuf], sem, add=True)
        pltpu.async_copy(aw, shared2.at[d8buf], sem, add=True)
        pltpu.make_async_copy(msgb, shared.at[dbuf], sem).wait()
        pltpu.make_async_copy(aw, shared2.at[d8buf], sem).wait()
        return carry

    lax.fori_loop(0, NCHUNK, chunk, 0)
    plsc.subcore_barrier()
    pltpu.sync_copy(shared.at[pl.ds(s * rows, rows)],
                    outp_hbm.at[pl.ds(c * NPAD + s * rows, rows)])
    pltpu.sync_copy(shared2.at[pl.ds(s * rows2, rows2)],
                    aesum_hbm.at[pl.ds(c * NPAD8 + s * rows2, rows2)])


def _sc_c(ae_pk, xpd, dst):
    kfn = pl.kernel(
        _scc_body,
        out_type=[jax.ShapeDtypeStruct((NC * NPAD, C), jnp.float32),
                  jax.ShapeDtypeStruct((NC * NPAD8, C), jnp.float32)],
        mesh=_mesh(),
        compiler_params=_SC_PARAMS,
        scratch_types=[
            pltpu.VMEM((G,), jnp.int32),
            pltpu.VMEM((G,), jnp.int32),
            pltpu.VMEM((16,), jnp.int32),
            pltpu.VMEM((16, C), jnp.float32),
            pltpu.VMEM((G, C), jnp.float32),
            pltpu.VMEM((G, C), jnp.float32),
            pltpu.VMEM((80, C), jnp.float32),
            pltpu.VMEM_SHARED((NPAD, C), jnp.float32),
            pltpu.VMEM_SHARED((NPAD8, C), jnp.float32),
            pltpu.SemaphoreType.DMA,
        ],
    )
    return kfn(ae_pk, xpd, dst)


def _pre_body(x_ref, win_ref, g1_ref, b1_ref, wsrc_ref, bsrc_ref,
              wdst_ref, bdst_ref, wlin_ref, blin_ref,
              qstat_ref, posg_ref, posb_ref,
              asrc_ref, adst_ref, xl_ref, pf_ref):
    x = x_ref[...]
    h = jnp.dot(x, win_ref[...], preferred_element_type=jnp.float32)
    m = jnp.mean(h, axis=0, keepdims=True)
    v = jnp.mean((h - m) ** 2, axis=0, keepdims=True)
    h = (h - m) / jnp.sqrt(v + EPS) * g1_ref[...] + b1_ref[...]
    h = jnp.maximum(h, 0.0)
    asrc_ref[...] = jnp.dot(h, wsrc_ref[...], preferred_element_type=jnp.float32) + bsrc_ref[...]
    adst_ref[...] = jnp.dot(h, wdst_ref[...], preferred_element_type=jnp.float32) + bdst_ref[...]
    xl_ref[...] = jnp.dot(h, wlin_ref[...], preferred_element_type=jnp.float32) + blin_ref[...]
    st = jnp.sum(qstat_ref[...], axis=(0, 1)).reshape(1, 128)
    lane = lax.broadcasted_iota(jnp.int32, (1, 128), 1)
    grpid = lane // 16
    sums = [jnp.sum(jnp.where(grpid == k, st, 0.0)) for k in range(6)]
    out_row = jnp.zeros((1, 128), jnp.float32)
    for k in range(3):
        mq = sums[k] / E
        vq = sums[k + 3] / E - mq * mq
        scale = posg_ref[0, k] / jnp.sqrt(vq + EPS)
        shift = posb_ref[0, k] - mq * scale
        out_row = jnp.where(lane == k, scale, out_row)
        out_row = jnp.where(lane == 3 + k, shift, out_row)
    pf_ref[...] = out_row


def _t3a_body(a_ref, bstat_ref, g1_ref, b1_ref, w1_ref, bb1_ref,
              t_ref, ts_ref, tq_ref):
    i = pl.program_id(0)
    st = jnp.sum(bstat_ref[...], axis=0)
    m = st[0:1] / E
    v = st[1:2] / E - m * m
    scale = g1_ref[...] / jnp.sqrt(v + EPS)
    shift = b1_ref[...] - m * scale
    a = jnp.maximum(a_ref[...] * scale + shift, 0.0)
    t = jnp.dot(a, w1_ref[...], preferred_element_type=jnp.float32) + bb1_ref[...]
    t_ref[...] = t

    @pl.when(i == 0)
    def _():
        ts_ref[...] = jnp.zeros_like(ts_ref)
        tq_ref[...] = jnp.zeros_like(tq_ref)

    ts_ref[...] += jnp.sum(t, axis=0, keepdims=True)
    tq_ref[...] += jnp.sum(t * t, axis=0, keepdims=True)


def _t3b_body(t_ref, ts_ref, tq_ref, g2_ref, b2_ref, w2_ref, bb2_ref, ae_ref):
    m = ts_ref[...] / E
    v = tq_ref[...] / E - m * m
    scale = g2_ref[...] / jnp.sqrt(v + EPS)
    shift = b2_ref[...] - m * scale
    t = jnp.maximum(t_ref[...] * scale + shift, 0.0)
    af = jnp.dot(t, w2_ref[...], preferred_element_type=jnp.float32) + bb2_ref[...]
    ae_ref[...] = jnp.exp(jnp.maximum(af, -60.0))


def _norm_body(outp_ref, asum_ref, o_ref):
    denom = asum_ref[0] + asum_ref[1] + 1e-16
    denom = jnp.concatenate([denom] * SP, axis=1)
    o_ref[...] = (outp_ref[0] + outp_ref[1]) / denom


def _post_body(o_ref, g2_ref, b2_ref, wout_ref, g3_ref, b3_ref,
               xskip_ref, y_ref):
    o = o_ref[...]
    m = jnp.mean(o, axis=0, keepdims=True)
    v = jnp.mean((o - m) ** 2, axis=0, keepdims=True)
    h = (o - m) / jnp.sqrt(v + EPS) * g2_ref[...] + b2_ref[...]
    h = jnp.maximum(h, 0.0)
    h = jnp.dot(h, wout_ref[...], preferred_element_type=jnp.float32)
    m = jnp.mean(h, axis=0, keepdims=True)
    v = jnp.mean((h - m) ** 2, axis=0, keepdims=True)
    h = (h - m) / jnp.sqrt(v + EPS) * g3_ref[...] + b3_ref[...]
    y_ref[...] = jnp.maximum(h + xskip_ref[...], 0.0)


def kernel(x, pos, edge_index, params):
    p = params
    src, dst = edge_index[0], edge_index[1]
    pos_flat = pos.reshape(-1)
    r2 = lambda a: a.reshape(1, -1)

    w1pack = jnp.pad(
        jnp.concatenate([p['pos_W1'], p['pos_b1'][None, :]], axis=0).T.reshape(-1),
        (0, 4))
    w2pack = jnp.concatenate([p['pos_W2'], p['pos_b2'][None, :]], axis=0)

    qstat = _sc_a(pos_flat, src, dst, w1pack)

    asrc, adst, xl, pf0 = pl.pallas_call(
        _pre_body,
        out_shape=[jax.ShapeDtypeStruct((N, C), jnp.float32)] * 3
        + [jax.ShapeDtypeStruct((1, 128), jnp.float32)],
    )(x, p['lin_in_W'], r2(p['bn1_g']), r2(p['bn1_b']),
      p['pt_src_W'], r2(p['pt_src_b']), p['pt_dst_W'], r2(p['pt_dst_b']),
      p['pt_lin_W'], r2(p['pt_lin_b']),
      qstat, r2(jnp.pad(p['pos_bn_g'], (0, 13))), r2(jnp.pad(p['pos_bn_b'], (0, 13))))

    pfpack = jnp.concatenate([pf0[:, 0:6], w1pack[None, 0:12],
                              jnp.zeros((1, 110), jnp.float32)], axis=1)

    a, xpd, bstat = _sc_b(asrc, adst, xl, pos_flat, src, dst, pfpack, w2pack)

    BE = 5000
    nb = E // BE
    t, ts, tq = pl.pallas_call(
        _t3a_body,
        grid=(nb,),
        in_specs=[
            pl.BlockSpec((BE, C), lambda i: (i, 0)),
            pl.BlockSpec((NW, 2, C), lambda i: (0, 0, 0)),
            pl.BlockSpec((1, C), lambda i: (0, 0)),
            pl.BlockSpec((1, C), lambda i: (0, 0)),
            pl.BlockSpec((C, CA), lambda i: (0, 0)),
            pl.BlockSpec((1, CA), lambda i: (0, 0)),
        ],
        out_specs=[
            pl.BlockSpec((BE, CA), lambda i: (i, 0)),
            pl.BlockSpec((1, CA), lambda i: (0, 0)),
            pl.BlockSpec((1, CA), lambda i: (0, 0)),
        ],
        out_shape=[jax.ShapeDtypeStruct((E, CA), jnp.float32),
                   jax.ShapeDtypeStruct((1, CA), jnp.float32),
                   jax.ShapeDtypeStruct((1, CA), jnp.float32)],
    )(a, bstat, r2(p['attn_bn1_g']), r2(p['attn_bn1_b']),
      p['attn_W1'], r2(p['attn_b1']))

    ae = pl.pallas_call(
        _t3b_body,
        grid=(nb,),
        in_specs=[
            pl.BlockSpec((BE, CA), lambda i: (i, 0)),
            pl.BlockSpec((1, CA), lambda i: (0, 0)),
            pl.BlockSpec((1, CA), lambda i: (0, 0)),
            pl.BlockSpec((1, CA), lambda i: (0, 0)),
            pl.BlockSpec((1, CA), lambda i: (0, 0)),
            pl.BlockSpec((CA, CA), lambda i: (0, 0)),
            pl.BlockSpec((1, CA), lambda i: (0, 0)),
        ],
        out_specs=pl.BlockSpec((BE, CA), lambda i: (i, 0)),
        out_shape=jax.ShapeDtypeStruct((E, CA), jnp.float32),
    )(t, ts, tq, r2(p['attn_bn2_g']), r2(p['attn_bn2_b']),
      p['attn_W2'], r2(p['attn_b2']))

    ae_pk = ae.reshape(E // 8, C)
    outp, aesum = _sc_c(ae_pk, xpd, dst)
    outp = outp.reshape(NC, NPAD, C)[:, :N, :]
    asum = aesum.reshape(NC, NPAD, CA)[:, :N, :]

    BN = 2000
    o = pl.pallas_call(
        _norm_body,
        grid=(N // BN,),
        in_specs=[
            pl.BlockSpec((NC, BN, C), lambda i: (0, i, 0)),
            pl.BlockSpec((NC, BN, CA), lambda i: (0, i, 0)),
        ],
        out_specs=pl.BlockSpec((BN, C), lambda i: (i, 0)),
        out_shape=jax.ShapeDtypeStruct((N, C), jnp.float32),
    )(outp, asum)

    y = pl.pallas_call(
        _post_body,
        out_shape=jax.ShapeDtypeStruct((N, C), jnp.float32),
    )(o, r2(p['bn2_g']), r2(p['bn2_b']), p['lin_out_W'],
      r2(p['bn3_g']), r2(p['bn3_b']), x)
    return y
